# baseline (speedup 1.0000x reference)
import functools

import jax
import jax.numpy as jnp
from jax import lax
from jax.experimental import pallas as pl
from jax.experimental.pallas import tpu as pltpu
from jax.experimental.pallas import tpu_sc as plsc

_NC = 2
_NS = 16
_NW = _NC * _NS
_LANES = 16


def _mlp_body(tab_ref, w1_ref, b1_ref, w2_ref, b2_ref, out_ref):
    h = jnp.dot(tab_ref[...], w1_ref[...], preferred_element_type=jnp.float32)
    h = jnp.maximum(h + b1_ref[...], 0.0)
    o = jnp.dot(h, w2_ref[...], preferred_element_type=jnp.float32)
    out_ref[...] = o + b2_ref[...]


def _mlp_table(table, W1, b1, W2, b2):
    V, D = table.shape
    return pl.pallas_call(
        _mlp_body,
        out_shape=jax.ShapeDtypeStruct((V, D), jnp.float32),
    )(table, W1, b1.reshape(1, D), W2, b2.reshape(1, D))


@functools.lru_cache(maxsize=None)
def _make_gather_t(V, D, B, H):
    assert B % (_NW * 128) == 0 and D % 8 == 0
    b_per_w = B // _NW
    n_kb = b_per_w // 128
    n_blocks = H * n_kb
    assert n_blocks % 2 == 0
    mesh = plsc.VectorSubcoreMesh(
        core_axis_name="c", subcore_axis_name="s",
        num_cores=_NC, num_subcores=_NS,
    )

    @functools.partial(
        pl.kernel,
        mesh=mesh,
        out_type=jax.ShapeDtypeStruct((H, D // 8, B // 128, 8, 128),
                                      jnp.float32),
        compiler_params=pltpu.CompilerParams(use_tc_tiling_on_sc=False,
                                             needs_layout_passes=False),
        scratch_types=[
            pltpu.VMEM((V * D,), jnp.float32),
            pltpu.VMEM((H * b_per_w,), jnp.int32),
            pltpu.VMEM((D, 128), jnp.float32),
            pltpu.VMEM((D, 128), jnp.float32),
            pltpu.SemaphoreType.DMA,
            pltpu.SemaphoreType.DMA,
            pltpu.SemaphoreType.DMA,
        ],
    )
    def gather(tab_hbm, idx_hbm, out_hbm, tab_v, idx_v, st0, st1,
               s_idx, so0, so1):
        wid = lax.axis_index("s") * _NC + lax.axis_index("c")
        b0 = wid * b_per_w
        st = (st0, st1)
        so = (so0, so1)

        for h in range(H):
            pltpu.async_copy(
                idx_hbm.at[pl.ds(h * B + b0, b_per_w)],
                idx_v.at[pl.ds(h * b_per_w, b_per_w)], s_idx)
        pltpu.sync_copy(tab_hbm, tab_v)
        for h in range(H):
            pltpu.make_async_copy(
                idx_hbm.at[pl.ds(h * B + b0, b_per_w)],
                idx_v.at[pl.ds(h * b_per_w, b_per_w)], s_idx).wait()

        kb0 = b0 // 128
        iot = lax.iota(jnp.int32, 16)

        def out_pairs(st_ref, h, kb):
            return [(st_ref.at[pl.ds(kd * 8, 8), pl.ds(0, 128)],
                     out_hbm.at[h, kd, kb0 + kb])
                    for kd in range(D // 8)]

        def block(t, par):
            h = t // n_kb
            kb = lax.rem(t, n_kb)
            st_ref = st[par]
            pairs = out_pairs(st_ref, h, kb)

            @pl.when(t >= 2)
            def _():
                for src, dsti in pairs:
                    pltpu.make_async_copy(src, dsti, so[par]).wait()

            ib = h * b_per_w + kb * 128
            st_q = [st_ref.at[pl.ds(q * _LANES, _LANES), pl.ds(0, 128)]
                    for q in range(D // _LANES)]
            for g in range(128 // _LANES):
                iv = idx_v[pl.ds(ib + g * _LANES, _LANES)]
                row_base = iv * D
                bcol = iot + (g * _LANES)
                for k in range(_LANES):
                    dkk = (iot + k) & (_LANES - 1)
                    la = row_base + dkk
                    vs = [plsc.load_gather(
                              tab_v, [la + (q * _LANES) if q else la])
                          for q in range(D // _LANES)]
                    for q in range(D // _LANES):
                        plsc.store_scatter(st_q[q], [dkk, bcol], vs[q])
            for src, dsti in pairs:
                pltpu.async_copy(src, dsti, so[par])

        def body(t2, carry):
            block(t2 * 2, 0)
            block(t2 * 2 + 1, 1)
            return carry

        lax.fori_loop(0, n_blocks // 2, body, 0)

        for par, st_ref in enumerate(st):
            for src, dsti in out_pairs(st_ref, H - 1, 0):
                pltpu.make_async_copy(src, dsti, so[par]).wait()

    return gather


def kernel(emotion_ids, table, W1, b1, W2, b2):
    Bb, H = emotion_ids.shape
    V, D = table.shape
    mlp_tab = _mlp_table(table, W1, b1, W2, b2)
    tab_flat = mlp_tab.reshape(-1)
    idx_t = emotion_ids.T.reshape(-1).astype(jnp.int32)
    out5 = _make_gather_t(V, D, Bb, H)(tab_flat, idx_t)
    return jnp.transpose(out5, (2, 4, 0, 1, 3)).reshape(Bb, H, D)


# speedup vs baseline: 1.0276x; 1.0276x over previous
import functools

import jax
import jax.numpy as jnp
from jax import lax
from jax.experimental import pallas as pl
from jax.experimental.pallas import tpu as pltpu
from jax.experimental.pallas import tpu_sc as plsc

_NC = 2
_NS = 16
_NW = _NC * _NS
_LANES = 16


def _mlp_body(tab_ref, w1_ref, b1_ref, w2_ref, b2_ref, out_ref):
    h = jnp.dot(tab_ref[...], w1_ref[...], preferred_element_type=jnp.float32)
    h = jnp.maximum(h + b1_ref[...], 0.0)
    o = jnp.dot(h, w2_ref[...], preferred_element_type=jnp.float32)
    out_ref[...] = o + b2_ref[...]


def _mlp_table(table, W1, b1, W2, b2):
    V, D = table.shape
    return pl.pallas_call(
        _mlp_body,
        out_shape=jax.ShapeDtypeStruct((V, D), jnp.float32),
    )(table, W1, b1.reshape(1, D), W2, b2.reshape(1, D))


@functools.lru_cache(maxsize=None)
def _make_gather_t(V, D, B, H):
    assert B % (_NW * 128) == 0 and D % 8 == 0
    b_per_w = B // _NW
    n_kb = b_per_w // 128
    n_blocks = H * n_kb
    assert n_blocks % 2 == 0
    mesh = plsc.VectorSubcoreMesh(
        core_axis_name="c", subcore_axis_name="s",
        num_cores=_NC, num_subcores=_NS,
    )

    @functools.partial(
        pl.kernel,
        mesh=mesh,
        out_type=jax.ShapeDtypeStruct((H, D // 8, B // 128, 1024),
                                      jnp.float32),
        compiler_params=pltpu.CompilerParams(use_tc_tiling_on_sc=False,
                                             needs_layout_passes=False),
        scratch_types=[
            pltpu.VMEM((V * D,), jnp.float32),
            pltpu.VMEM((H * b_per_w,), jnp.int32),
            pltpu.VMEM((D * 128,), jnp.float32),
            pltpu.VMEM((D * 128,), jnp.float32),
            pltpu.SemaphoreType.DMA,
            pltpu.SemaphoreType.DMA,
            pltpu.SemaphoreType.DMA,
        ],
    )
    def gather(tab_hbm, idx_hbm, out_hbm, tab_v, idx_v, st0, st1,
               s_idx, so0, so1):
        wid = lax.axis_index("s") * _NC + lax.axis_index("c")
        b0 = wid * b_per_w
        st = (st0, st1)
        so = (so0, so1)

        for h in range(H):
            pltpu.async_copy(
                idx_hbm.at[pl.ds(h * B + b0, b_per_w)],
                idx_v.at[pl.ds(h * b_per_w, b_per_w)], s_idx)
        pltpu.sync_copy(tab_hbm, tab_v)
        for h in range(H):
            pltpu.make_async_copy(
                idx_hbm.at[pl.ds(h * B + b0, b_per_w)],
                idx_v.at[pl.ds(h * b_per_w, b_per_w)], s_idx).wait()

        kb0 = b0 // 128
        iot = lax.iota(jnp.int32, 16)

        def out_pairs(st_ref, h, kb):
            return [(st_ref.at[pl.ds(kd * 1024, 1024)],
                     out_hbm.at[h, kd, kb0 + kb])
                    for kd in range(D // 8)]

        def block(t, par):
            h = t // n_kb
            kb = lax.rem(t, n_kb)
            st_ref = st[par]
            pairs = out_pairs(st_ref, h, kb)

            @pl.when(t >= 2)
            def _():
                for src, dsti in pairs:
                    pltpu.make_async_copy(src, dsti, so[par]).wait()

            ib = h * b_per_w + kb * 128
            st_q = [st_ref.at[pl.ds(q * 2048, 2048)]
                    for q in range(D // _LANES)]
            for g in range(128 // _LANES):
                iv = idx_v[pl.ds(ib + g * _LANES, _LANES)]
                row_base = iv * D
                bcol = iot + (g * _LANES)
                for k in range(_LANES):
                    dkk = (iot + k) & (_LANES - 1)
                    la = row_base + dkk
                    f = dkk * 128 + bcol
                    vs = [plsc.load_gather(
                              tab_v, [la + (q * _LANES) if q else la])
                          for q in range(D // _LANES)]
                    for q in range(D // _LANES):
                        plsc.store_scatter(st_q[q], [f], vs[q])
            for src, dsti in pairs:
                pltpu.async_copy(src, dsti, so[par])

        def body(t2, carry):
            block(t2 * 2, 0)
            block(t2 * 2 + 1, 1)
            return carry

        lax.fori_loop(0, n_blocks // 2, body, 0)

        for par, st_ref in enumerate(st):
            for src, dsti in out_pairs(st_ref, H - 1, 0):
                pltpu.make_async_copy(src, dsti, so[par]).wait()

    return gather


def kernel(emotion_ids, table, W1, b1, W2, b2):
    Bb, H = emotion_ids.shape
    V, D = table.shape
    mlp_tab = _mlp_table(table, W1, b1, W2, b2)
    tab_flat = mlp_tab.reshape(-1)
    idx_t = emotion_ids.T.reshape(-1).astype(jnp.int32)
    out4 = _make_gather_t(V, D, Bb, H)(tab_flat, idx_t)
    out5 = out4.reshape(H, D // 8, Bb // 128, 8, 128)
    return jnp.transpose(out5, (2, 4, 0, 1, 3)).reshape(Bb, H, D)


# speedup vs baseline: 1.3285x; 1.2929x over previous
import functools

import jax
import jax.numpy as jnp
from jax import lax
from jax.experimental import pallas as pl
from jax.experimental.pallas import tpu as pltpu
from jax.experimental.pallas import tpu_sc as plsc

_NC = 2
_NS = 16
_NW = _NC * _NS
_LANES = 16


def _mlp_body(tab_ref, w1_ref, b1_ref, w2_ref, b2_ref, out_ref):
    h = jnp.dot(tab_ref[...], w1_ref[...], preferred_element_type=jnp.float32)
    h = jnp.maximum(h + b1_ref[...], 0.0)
    o = jnp.dot(h, w2_ref[...], preferred_element_type=jnp.float32)
    out_ref[...] = o + b2_ref[...]


def _mlp_table(table, W1, b1, W2, b2):
    V, D = table.shape
    return pl.pallas_call(
        _mlp_body,
        out_shape=jax.ShapeDtypeStruct((V, D), jnp.float32),
    )(table, W1, b1.reshape(1, D), W2, b2.reshape(1, D))


@functools.lru_cache(maxsize=None)
def _make_gather_t(V, D, B, H):
    assert B % (_NW * 128) == 0 and D % 8 == 0
    b_per_w = B // _NW
    n_kb = b_per_w // 128
    n_blocks = H * n_kb
    assert n_blocks % 2 == 0
    mesh = plsc.VectorSubcoreMesh(
        core_axis_name="c", subcore_axis_name="s",
        num_cores=_NC, num_subcores=_NS,
    )

    @functools.partial(
        pl.kernel,
        mesh=mesh,
        out_type=jax.ShapeDtypeStruct((H, D // 8, B // 128, 1024),
                                      jnp.float32),
        compiler_params=pltpu.CompilerParams(use_tc_tiling_on_sc=False,
                                             needs_layout_passes=False),
        scratch_types=[
            pltpu.VMEM((V * D,), jnp.float32),
            pltpu.VMEM((H * b_per_w,), jnp.int32),
            pltpu.VMEM((D * 128,), jnp.float32),
            pltpu.VMEM((D * 128,), jnp.float32),
            pltpu.SemaphoreType.DMA,
            pltpu.SemaphoreType.DMA,
            pltpu.SemaphoreType.DMA,
        ],
    )
    def gather(tab_hbm, idx_hbm, out_hbm, tab_v, idx_v, st0, st1,
               s_idx, so0, so1):
        wid = lax.axis_index("s") * _NC + lax.axis_index("c")
        b0 = wid * b_per_w
        st = (st0, st1)
        so = (so0, so1)

        for h in range(H):
            pltpu.async_copy(
                idx_hbm.at[pl.ds(h * B + b0, b_per_w)],
                idx_v.at[pl.ds(h * b_per_w, b_per_w)], s_idx)
        pltpu.sync_copy(tab_hbm, tab_v)
        for h in range(H):
            pltpu.make_async_copy(
                idx_hbm.at[pl.ds(h * B + b0, b_per_w)],
                idx_v.at[pl.ds(h * b_per_w, b_per_w)], s_idx).wait()

        kb0 = b0 // 128
        iot = lax.iota(jnp.int32, 16)

        def out_pairs(st_ref, h, kb):
            return [(st_ref.at[pl.ds(kd * 1024, 1024)],
                     out_hbm.at[h, kd, kb0 + kb])
                    for kd in range(D // 8)]

        def block(t, par):
            h = t // n_kb
            kb = lax.rem(t, n_kb)
            st_ref = st[par]
            pairs = out_pairs(st_ref, h, kb)

            @pl.when(t >= 2)
            def _():
                for src, dsti in pairs:
                    pltpu.make_async_copy(src, dsti, so[par]).wait()

            ib = h * b_per_w + kb * 128
            st_q = [st_ref.at[pl.ds(q * 2048, 2048)]
                    for q in range(D // _LANES)]

            def kbody(k, carry):
                dkk = (iot + k) & (_LANES - 1)
                f0 = dkk * 128 + iot
                for g in range(128 // _LANES):
                    iv = idx_v[pl.ds(ib + g * _LANES, _LANES)]
                    la = iv * D + dkk
                    f = f0 + (g * _LANES) if g else f0
                    vs = [plsc.load_gather(
                              tab_v, [la + (q * _LANES) if q else la])
                          for q in range(D // _LANES)]
                    for q in range(D // _LANES):
                        plsc.store_scatter(st_q[q], [f], vs[q])
                return carry

            lax.fori_loop(0, _LANES, kbody, 0)
            for src, dsti in pairs:
                pltpu.async_copy(src, dsti, so[par])

        def body(t2, carry):
            block(t2 * 2, 0)
            block(t2 * 2 + 1, 1)
            return carry

        lax.fori_loop(0, n_blocks // 2, body, 0)

        for par, st_ref in enumerate(st):
            for src, dsti in out_pairs(st_ref, H - 1, 0):
                pltpu.make_async_copy(src, dsti, so[par]).wait()

    return gather


def kernel(emotion_ids, table, W1, b1, W2, b2):
    Bb, H = emotion_ids.shape
    V, D = table.shape
    mlp_tab = _mlp_table(table, W1, b1, W2, b2)
    tab_flat = mlp_tab.reshape(-1)
    idx_t = emotion_ids.T.reshape(-1).astype(jnp.int32)
    out4 = _make_gather_t(V, D, Bb, H)(tab_flat, idx_t)
    out5 = out4.reshape(H, D // 8, Bb // 128, 8, 128)
    return jnp.transpose(out5, (2, 4, 0, 1, 3)).reshape(Bb, H, D)


# speedup vs baseline: 1.8276x; 1.3757x over previous
import functools

import jax
import jax.numpy as jnp
from jax import lax
from jax.experimental import pallas as pl
from jax.experimental.pallas import tpu as pltpu
from jax.experimental.pallas import tpu_sc as plsc

_NC = 2
_NS = 16
_NW = _NC * _NS
_LANES = 16


def _mlp_body(tab_ref, w1_ref, b1_ref, w2_ref, b2_ref, out_ref):
    h = jnp.dot(tab_ref[...], w1_ref[...], preferred_element_type=jnp.float32)
    h = jnp.maximum(h + b1_ref[...], 0.0)
    o = jnp.dot(h, w2_ref[...], preferred_element_type=jnp.float32)
    out_ref[...] = o + b2_ref[...]


def _mlp_table(table, W1, b1, W2, b2):
    V, D = table.shape
    return pl.pallas_call(
        _mlp_body,
        out_shape=jax.ShapeDtypeStruct((V, D), jnp.float32),
    )(table, W1, b1.reshape(1, D), W2, b2.reshape(1, D))


@functools.lru_cache(maxsize=None)
def _make_gather_t(V, D, B, H):
    assert B % (_NW * 128) == 0 and D % 8 == 0
    b_per_w = B // _NW
    n_kb = b_per_w // 128
    n_blocks = H * n_kb
    assert n_blocks % 2 == 0
    mesh = plsc.VectorSubcoreMesh(
        core_axis_name="c", subcore_axis_name="s",
        num_cores=_NC, num_subcores=_NS,
    )

    @functools.partial(
        pl.kernel,
        mesh=mesh,
        out_type=jax.ShapeDtypeStruct((H, D // 8, B // 128, 1024),
                                      jnp.float32),
        compiler_params=pltpu.CompilerParams(use_tc_tiling_on_sc=False,
                                             needs_layout_passes=False),
        scratch_types=[
            pltpu.VMEM((V * D,), jnp.float32),
            pltpu.VMEM((H * b_per_w,), jnp.int32),
            pltpu.VMEM((D * 128,), jnp.float32),
            pltpu.VMEM((D * 128,), jnp.float32),
            pltpu.SemaphoreType.DMA,
            pltpu.SemaphoreType.DMA,
            pltpu.SemaphoreType.DMA,
        ],
    )
    def gather(tab_hbm, idx_hbm, out_hbm, tab_v, idx_v, st0, st1,
               s_idx, so0, so1):
        wid = lax.axis_index("s") * _NC + lax.axis_index("c")
        b0 = wid * b_per_w
        st = (st0, st1)
        so = (so0, so1)

        for h in range(H):
            pltpu.async_copy(
                idx_hbm.at[pl.ds(h * B + b0, b_per_w)],
                idx_v.at[pl.ds(h * b_per_w, b_per_w)], s_idx)
        pltpu.sync_copy(tab_hbm, tab_v)
        for h in range(H):
            pltpu.make_async_copy(
                idx_hbm.at[pl.ds(h * B + b0, b_per_w)],
                idx_v.at[pl.ds(h * b_per_w, b_per_w)], s_idx).wait()

        kb0 = b0 // 128
        iot = lax.iota(jnp.int32, 16)

        def out_pairs(st_ref, h, kb):
            return [(st_ref.at[pl.ds(kd * 1024, 1024)],
                     out_hbm.at[h, kd, kb0 + kb])
                    for kd in range(D // 8)]

        def block(t, par):
            h = t // n_kb
            kb = lax.rem(t, n_kb)
            st_ref = st[par]
            pairs = out_pairs(st_ref, h, kb)

            @pl.when(t >= 2)
            def _():
                for src, dsti in pairs:
                    pltpu.make_async_copy(src, dsti, so[par]).wait()

            ib = h * b_per_w + kb * 128
            st_q = [st_ref.at[pl.ds(q * 2048, 2048)]
                    for q in range(D // _LANES)]

            def kbody(k, carry):
                dkk = (iot + k) & (_LANES - 1)
                f0 = dkk * 128 + iot
                for gp in range(128 // _LANES // 2):
                    ga, gb = 2 * gp, 2 * gp + 1
                    iva = idx_v[pl.ds(ib + ga * _LANES, _LANES)]
                    ivb = idx_v[pl.ds(ib + gb * _LANES, _LANES)]
                    laa = iva * D + dkk
                    lab = ivb * D + dkk
                    fa = f0 + (ga * _LANES) if ga else f0
                    fb = f0 + (gb * _LANES)
                    vsa = [plsc.load_gather(
                               tab_v, [laa + (q * _LANES) if q else laa])
                           for q in range(D // _LANES)]
                    vsb = [plsc.load_gather(
                               tab_v, [lab + (q * _LANES) if q else lab])
                           for q in range(D // _LANES)]
                    for q in range(D // _LANES):
                        plsc.store_scatter(st_q[q], [fa], vsa[q])
                        plsc.store_scatter(st_q[q], [fb], vsb[q])
                return carry

            lax.fori_loop(0, _LANES, kbody, 0)
            for src, dsti in pairs:
                pltpu.async_copy(src, dsti, so[par])

        def body(t2, carry):
            block(t2 * 2, 0)
            block(t2 * 2 + 1, 1)
            return carry

        lax.fori_loop(0, n_blocks // 2, body, 0)

        for par, st_ref in enumerate(st):
            for src, dsti in out_pairs(st_ref, H - 1, 0):
                pltpu.make_async_copy(src, dsti, so[par]).wait()

    return gather


def kernel(emotion_ids, table, W1, b1, W2, b2):
    Bb, H = emotion_ids.shape
    V, D = table.shape
    mlp_tab = _mlp_table(table, W1, b1, W2, b2)
    tab_flat = mlp_tab.reshape(-1)
    idx_t = emotion_ids.T.reshape(-1).astype(jnp.int32)
    out4 = _make_gather_t(V, D, Bb, H)(tab_flat, idx_t)
    out5 = out4.reshape(H, D // 8, Bb // 128, 8, 128)
    return jnp.transpose(out5, (2, 4, 0, 1, 3)).reshape(Bb, H, D)


# speedup vs baseline: 2.1241x; 1.1622x over previous
import functools

import jax
import jax.numpy as jnp
from jax import lax
from jax.experimental import pallas as pl
from jax.experimental.pallas import tpu as pltpu
from jax.experimental.pallas import tpu_sc as plsc

_NC = 2
_NS = 16
_NW = _NC * _NS
_LANES = 16


def _mlp_body(tab_ref, w1_ref, b1_ref, w2_ref, b2_ref, out_ref):
    h = jnp.dot(tab_ref[...], w1_ref[...], preferred_element_type=jnp.float32)
    h = jnp.maximum(h + b1_ref[...], 0.0)
    o = jnp.dot(h, w2_ref[...], preferred_element_type=jnp.float32)
    out_ref[...] = o + b2_ref[...]


def _mlp_table(table, W1, b1, W2, b2):
    V, D = table.shape
    return pl.pallas_call(
        _mlp_body,
        out_shape=jax.ShapeDtypeStruct((V, D), jnp.float32),
    )(table, W1, b1.reshape(1, D), W2, b2.reshape(1, D))


@functools.lru_cache(maxsize=None)
def _make_gather_t(V, D, B, H):
    assert B % (_NW * 128) == 0 and D % 8 == 0
    b_per_w = B // _NW
    n_kb = b_per_w // 128
    n_blocks = H * n_kb
    assert n_blocks % 2 == 0
    mesh = plsc.VectorSubcoreMesh(
        core_axis_name="c", subcore_axis_name="s",
        num_cores=_NC, num_subcores=_NS,
    )

    @functools.partial(
        pl.kernel,
        mesh=mesh,
        out_type=jax.ShapeDtypeStruct((H, D // 8, B // 128, 1024),
                                      jnp.float32),
        compiler_params=pltpu.CompilerParams(use_tc_tiling_on_sc=False,
                                             needs_layout_passes=False),
        scratch_types=[
            pltpu.VMEM((V * D,), jnp.float32),
            pltpu.VMEM((H * b_per_w,), jnp.int32),
            pltpu.VMEM((D * 128,), jnp.float32),
            pltpu.VMEM((D * 128,), jnp.float32),
            pltpu.SemaphoreType.DMA,
            pltpu.SemaphoreType.DMA,
            pltpu.SemaphoreType.DMA,
        ],
    )
    def gather(tab_hbm, idx_hbm, out_hbm, tab_v, idx_v, st0, st1,
               s_idx, so0, so1):
        wid = lax.axis_index("s") * _NC + lax.axis_index("c")
        b0 = wid * b_per_w
        st = (st0, st1)
        so = (so0, so1)

        for h in range(H):
            pltpu.async_copy(
                idx_hbm.at[pl.ds(h * B + b0, b_per_w)],
                idx_v.at[pl.ds(h * b_per_w, b_per_w)], s_idx)
        pltpu.sync_copy(tab_hbm, tab_v)
        for h in range(H):
            pltpu.make_async_copy(
                idx_hbm.at[pl.ds(h * B + b0, b_per_w)],
                idx_v.at[pl.ds(h * b_per_w, b_per_w)], s_idx).wait()

        kb0 = b0 // 128
        iot = lax.iota(jnp.int32, 16)

        def out_pairs(st_ref, h, kb):
            return [(st_ref.at[pl.ds(kd * 1024, 1024)],
                     out_hbm.at[h, kd, kb0 + kb])
                    for kd in range(D // 8)]

        def block(t, par):
            h = t // n_kb
            kb = lax.rem(t, n_kb)
            st_ref = st[par]
            pairs = out_pairs(st_ref, h, kb)

            @pl.when(t >= 2)
            def _():
                for src, dsti in pairs:
                    pltpu.make_async_copy(src, dsti, so[par]).wait()

            ib = h * b_per_w + kb * 128
            st_q = [st_ref.at[pl.ds(q * 2048, 2048)]
                    for q in range(D // _LANES)]

            def kbody(k, carry):
                dkk = (iot + k) & (_LANES - 1)
                f0 = dkk * 128 + iot
                for gp in range(128 // _LANES // 4):
                    gs = [4 * gp + i for i in range(4)]
                    ivs = [idx_v[pl.ds(ib + g * _LANES, _LANES)]
                           for g in gs]
                    las = [iv * D + dkk for iv in ivs]
                    fs = [f0 + (g * _LANES) if g else f0 for g in gs]
                    vss = [[plsc.load_gather(
                                tab_v, [la + (q * _LANES) if q else la])
                            for q in range(D // _LANES)]
                           for la in las]
                    for q in range(D // _LANES):
                        for c in range(4):
                            plsc.store_scatter(st_q[q], [fs[c]], vss[c][q])
                return carry

            lax.fori_loop(0, _LANES, kbody, 0)
            for src, dsti in pairs:
                pltpu.async_copy(src, dsti, so[par])

        def body(t2, carry):
            block(t2 * 2, 0)
            block(t2 * 2 + 1, 1)
            return carry

        lax.fori_loop(0, n_blocks // 2, body, 0)

        for par, st_ref in enumerate(st):
            for src, dsti in out_pairs(st_ref, H - 1, 0):
                pltpu.make_async_copy(src, dsti, so[par]).wait()

    return gather


def kernel(emotion_ids, table, W1, b1, W2, b2):
    Bb, H = emotion_ids.shape
    V, D = table.shape
    mlp_tab = _mlp_table(table, W1, b1, W2, b2)
    tab_flat = mlp_tab.reshape(-1)
    idx_t = emotion_ids.T.reshape(-1).astype(jnp.int32)
    out4 = _make_gather_t(V, D, Bb, H)(tab_flat, idx_t)
    out5 = out4.reshape(H, D // 8, Bb // 128, 8, 128)
    return jnp.transpose(out5, (2, 4, 0, 1, 3)).reshape(Bb, H, D)


# speedup vs baseline: 2.3149x; 1.0898x over previous
import functools

import jax
import jax.numpy as jnp
from jax import lax
from jax.experimental import pallas as pl
from jax.experimental.pallas import tpu as pltpu
from jax.experimental.pallas import tpu_sc as plsc

_NC = 2
_NS = 16
_NW = _NC * _NS
_LANES = 16


def _mlp_body(tab_ref, w1_ref, b1_ref, w2_ref, b2_ref, out_ref):
    h = jnp.dot(tab_ref[...], w1_ref[...], preferred_element_type=jnp.float32)
    h = jnp.maximum(h + b1_ref[...], 0.0)
    o = jnp.dot(h, w2_ref[...], preferred_element_type=jnp.float32)
    out_ref[...] = o + b2_ref[...]


def _mlp_table(table, W1, b1, W2, b2):
    V, D = table.shape
    return pl.pallas_call(
        _mlp_body,
        out_shape=jax.ShapeDtypeStruct((V, D), jnp.float32),
    )(table, W1, b1.reshape(1, D), W2, b2.reshape(1, D))


@functools.lru_cache(maxsize=None)
def _make_gather_t(V, D, B, H):
    assert B % (_NW * 128) == 0 and D % 8 == 0
    b_per_w = B // _NW
    n_kb = b_per_w // 128
    n_blocks = H * n_kb
    assert n_blocks % 2 == 0
    mesh = plsc.VectorSubcoreMesh(
        core_axis_name="c", subcore_axis_name="s",
        num_cores=_NC, num_subcores=_NS,
    )

    @functools.partial(
        pl.kernel,
        mesh=mesh,
        out_type=jax.ShapeDtypeStruct((H, D // 8, B // 128, 1024),
                                      jnp.float32),
        compiler_params=pltpu.CompilerParams(use_tc_tiling_on_sc=False,
                                             needs_layout_passes=False),
        scratch_types=[
            pltpu.VMEM((V * D,), jnp.float32),
            pltpu.VMEM((H * b_per_w,), jnp.int32),
            pltpu.VMEM((D * 128,), jnp.float32),
            pltpu.VMEM((D * 128,), jnp.float32),
            pltpu.SemaphoreType.DMA,
            pltpu.SemaphoreType.DMA,
            pltpu.SemaphoreType.DMA,
        ],
    )
    def gather(tab_hbm, idx_hbm, out_hbm, tab_v, idx_v, st0, st1,
               s_idx, so0, so1):
        wid = lax.axis_index("s") * _NC + lax.axis_index("c")
        b0 = wid * b_per_w
        st = (st0, st1)
        so = (so0, so1)

        for h in range(H):
            pltpu.async_copy(
                idx_hbm.at[pl.ds(h * B + b0, b_per_w)],
                idx_v.at[pl.ds(h * b_per_w, b_per_w)], s_idx)
        pltpu.sync_copy(tab_hbm, tab_v)
        for h in range(H):
            pltpu.make_async_copy(
                idx_hbm.at[pl.ds(h * B + b0, b_per_w)],
                idx_v.at[pl.ds(h * b_per_w, b_per_w)], s_idx).wait()

        kb0 = b0 // 128
        iot = lax.iota(jnp.int32, 16)

        def out_pairs(st_ref, h, kb):
            return [(st_ref.at[pl.ds(kd * 1024, 1024)],
                     out_hbm.at[h, kd, kb0 + kb])
                    for kd in range(D // 8)]

        def block(t, par):
            h = t // n_kb
            kb = lax.rem(t, n_kb)
            st_ref = st[par]
            pairs = out_pairs(st_ref, h, kb)

            @pl.when(t >= 2)
            def _():
                for src, dsti in pairs:
                    pltpu.make_async_copy(src, dsti, so[par]).wait()

            ib = h * b_per_w + kb * 128
            st_q = [st_ref.at[pl.ds(q * 2048, 2048)]
                    for q in range(D // _LANES)]

            def kbody(k, carry):
                dkk = (iot + k) & (_LANES - 1)
                f0 = dkk * 128 + iot
                for gp in range(128 // _LANES // 8):
                    gs = [8 * gp + i for i in range(8)]
                    ivs = [idx_v[pl.ds(ib + g * _LANES, _LANES)]
                           for g in gs]
                    las = [iv * D + dkk for iv in ivs]
                    fs = [f0 + (g * _LANES) if g else f0 for g in gs]
                    vss = [[plsc.load_gather(
                                tab_v, [la + (q * _LANES) if q else la])
                            for q in range(D // _LANES)]
                           for la in las]
                    for q in range(D // _LANES):
                        for c in range(len(gs)):
                            plsc.store_scatter(st_q[q], [fs[c]], vss[c][q])
                return carry

            lax.fori_loop(0, _LANES, kbody, 0)
            for src, dsti in pairs:
                pltpu.async_copy(src, dsti, so[par])

        def body(t2, carry):
            block(t2 * 2, 0)
            block(t2 * 2 + 1, 1)
            return carry

        lax.fori_loop(0, n_blocks // 2, body, 0)

        for par, st_ref in enumerate(st):
            for src, dsti in out_pairs(st_ref, H - 1, 0):
                pltpu.make_async_copy(src, dsti, so[par]).wait()

    return gather


def kernel(emotion_ids, table, W1, b1, W2, b2):
    Bb, H = emotion_ids.shape
    V, D = table.shape
    mlp_tab = _mlp_table(table, W1, b1, W2, b2)
    tab_flat = mlp_tab.reshape(-1)
    idx_t = emotion_ids.T.reshape(-1).astype(jnp.int32)
    out4 = _make_gather_t(V, D, Bb, H)(tab_flat, idx_t)
    out5 = out4.reshape(H, D // 8, Bb // 128, 8, 128)
    return jnp.transpose(out5, (2, 4, 0, 1, 3)).reshape(Bb, H, D)
